# baseline (speedup 1.0000x reference)
import functools

import jax
import jax.numpy as jnp
from jax import lax
from jax.experimental import pallas as pl
from jax.experimental.pallas import tpu as pltpu
from jax.experimental.pallas import tpu_sc as plsc

N = 10000
E = 320000
D = 128
NC = 2
NS = 16
K = 128
NP = 10240
ER = 2560
EP = ER * K
SROWS = ER // NS
NBR = 16
NBATCH = SROWS // NBR
RPW = NP // NS

_mesh = plsc.VectorSubcoreMesh(core_axis_name="c", subcore_axis_name="s")


@functools.partial(
    pl.kernel,
    out_type=(
        jax.ShapeDtypeStruct((NC, NP, D), jnp.float32),
        jax.ShapeDtypeStruct((NC, NP, D), jnp.float32),
    ),
    mesh=_mesh,
    scratch_types=(
        pltpu.VMEM_SHARED((NP, D), jnp.float32),
        pltpu.VMEM((NBR, K), jnp.int32),
        pltpu.VMEM((NBR, K), jnp.int32),
        pltpu.VMEM((K, D), jnp.float32),
        pltpu.SemaphoreType.DMA,
    ),
)
def _segment_sums(srcs2d, dsts2d, lbls, zrows, ones_in,
                  out_sums, out_cnts,
                  acc_sh, src_b, dst_b, rows_v, sem):
    c = lax.axis_index("c")
    s = lax.axis_index("s")
    r0 = pl.multiple_of(s * RPW, 8)

    def zero_acc():
        pltpu.sync_copy(zrows, acc_sh.at[pl.ds(r0, RPW)])

    def writeout(dst_hbm):
        pltpu.sync_copy(acc_sh.at[pl.ds(r0, RPW)], dst_hbm.at[c, pl.ds(r0, RPW)])

    zero_acc()
    plsc.subcore_barrier()

    def batch_a(b, carry):
        row0 = pl.multiple_of(c * ER + s * SROWS + b * NBR, 8)
        pltpu.sync_copy(srcs2d.at[pl.ds(row0, NBR)], src_b)
        pltpu.sync_copy(dsts2d.at[pl.ds(row0, NBR)], dst_b)
        for j in range(NBR):
            pltpu.async_copy(lbls.at[src_b.at[j]], rows_v, sem).wait()
            pltpu.sync_copy(rows_v, acc_sh.at[dst_b.at[j]], add=True)
        return carry

    lax.fori_loop(0, NBATCH, batch_a, 0)
    plsc.subcore_barrier()
    writeout(out_sums)
    plsc.subcore_barrier()

    zero_acc()
    pltpu.sync_copy(ones_in, rows_v)
    plsc.subcore_barrier()

    def batch_b(b, carry):
        row0 = pl.multiple_of(c * ER + s * SROWS + b * NBR, 8)
        pltpu.sync_copy(dsts2d.at[pl.ds(row0, NBR)], dst_b)
        for j in range(NBR):
            pltpu.sync_copy(rows_v, acc_sh.at[dst_b.at[j]], add=True)
        return carry

    lax.fori_loop(0, NBATCH, batch_b, 0)
    plsc.subcore_barrier()
    writeout(out_cnts)


_BR = 1000


def _combine_body(lbls_ref, mask_ref, s1_ref, c1_ref, s2_ref, c2_ref, o_ref):
    c1 = jnp.maximum(c1_ref[:, 0:1], 1.0)
    c2 = jnp.maximum(c2_ref[:, 0:1], 1.0)
    mean = (s1_ref[...] / c1 + s2_ref[...] / c2) * 0.5
    o_ref[...] = jnp.where(mask_ref[...] > 0, mean, lbls_ref[...])


def _pad_set(knn):
    npad = EP - E
    psrc = jnp.zeros((npad,), jnp.int32)
    pdst = N + (jnp.arange(npad, dtype=jnp.int32) % (NP - N))
    src = jnp.concatenate([knn[0], psrc])
    dst = jnp.concatenate([knn[1], pdst])
    return src, dst


def kernel(lbls, no_lbl_idx, knn_sc, knn_fc):
    s1, d1 = _pad_set(knn_sc)
    s2, d2 = _pad_set(knn_fc)
    srcs2d = jnp.concatenate([s1, s2]).reshape(NC * ER, K)
    dsts2d = jnp.concatenate([d1, d2]).reshape(NC * ER, K)
    zrows = jnp.zeros((RPW, D), jnp.float32)
    ones = jnp.ones((K, D), jnp.float32)
    sums, cnts = _segment_sums(srcs2d, dsts2d, lbls, zrows, ones)

    mask2d = no_lbl_idx.astype(jnp.int32).reshape(N, 1)
    return pl.pallas_call(
        _combine_body,
        out_shape=jax.ShapeDtypeStruct((N, D), jnp.float32),
        grid=(N // _BR,),
        in_specs=[
            pl.BlockSpec((_BR, D), lambda i: (i, 0)),
            pl.BlockSpec((_BR, 1), lambda i: (i, 0)),
            pl.BlockSpec((_BR, D), lambda i: (i, 0)),
            pl.BlockSpec((_BR, D), lambda i: (i, 0)),
            pl.BlockSpec((_BR, D), lambda i: (i, 0)),
            pl.BlockSpec((_BR, D), lambda i: (i, 0)),
        ],
        out_specs=pl.BlockSpec((_BR, D), lambda i: (i, 0)),
    )(lbls, mask2d, sums[0], cnts[0], sums[1], cnts[1])


# speedup vs baseline: 1.2140x; 1.2140x over previous
import functools

import jax
import jax.numpy as jnp
from jax import lax
from jax.experimental import pallas as pl
from jax.experimental.pallas import tpu as pltpu
from jax.experimental.pallas import tpu_sc as plsc

N = 10000
E = 320000
D = 128
NC = 2
NS = 16
K = 40
NP = 10240
EPW = E // NS
NCHUNK = EPW // K
RPW = NP // NS

_mesh = plsc.VectorSubcoreMesh(core_axis_name="c", subcore_axis_name="s")


@functools.partial(
    pl.kernel,
    out_type=(
        jax.ShapeDtypeStruct((NC, NP, D), jnp.float32),
        jax.ShapeDtypeStruct((NC, NP, D), jnp.float32),
    ),
    mesh=_mesh,
    scratch_types=(
        pltpu.VMEM_SHARED((NP, D), jnp.float32),
        pltpu.VMEM((K,), jnp.int32),
        pltpu.VMEM((K,), jnp.int32),
        pltpu.VMEM((K,), jnp.int32),
        pltpu.VMEM((K,), jnp.int32),
        pltpu.VMEM((K, D), jnp.float32),
        pltpu.VMEM((K, D), jnp.float32),
        pltpu.SemaphoreType.DMA,
        pltpu.SemaphoreType.DMA,
    ),
)
def _segment_sums(srcs, dsts, lbls, zrows, ones_in,
                  out_sums, out_cnts,
                  acc_sh, src0, src1, dst0, dst1, rows0, rows1, sem0, sem1):
    c = lax.axis_index("c")
    s = lax.axis_index("s")
    r0 = pl.multiple_of(s * RPW, 8)
    e0 = c * E + s * EPW
    srcb = (src0, src1)
    dstb = (dst0, dst1)
    rowsb = (rows0, rows1)
    semb = (sem0, sem1)

    def eoff(g):
        return pl.multiple_of(e0 + g * K, 8)

    def load_idx(g, b, with_src=True):
        if with_src:
            pltpu.sync_copy(srcs.at[pl.ds(eoff(g), K)], srcb[b])
        pltpu.sync_copy(dsts.at[pl.ds(eoff(g), K)], dstb[b])

    def start_gather(b):
        pltpu.async_copy(lbls.at[srcb[b]], rowsb[b], semb[b])

    def finish_scatter(b):
        pltpu.make_async_copy(lbls.at[srcb[b]], rowsb[b], semb[b]).wait()
        pltpu.sync_copy(rowsb[b], acc_sh.at[dstb[b]], add=True)

    pltpu.sync_copy(zrows, acc_sh.at[pl.ds(r0, RPW)])
    plsc.subcore_barrier()

    load_idx(0, 0)
    start_gather(0)

    def batch_a(i, carry):
        g = 2 * i + 1
        load_idx(g, 1)
        start_gather(1)
        finish_scatter(0)
        load_idx(g + 1, 0)
        start_gather(0)
        finish_scatter(1)
        return carry

    lax.fori_loop(0, (NCHUNK - 2) // 2, batch_a, 0)
    load_idx(NCHUNK - 1, 1)
    start_gather(1)
    finish_scatter(0)
    finish_scatter(1)
    plsc.subcore_barrier()
    pltpu.sync_copy(acc_sh.at[pl.ds(r0, RPW)], out_sums.at[c, pl.ds(r0, RPW)])
    plsc.subcore_barrier()

    pltpu.sync_copy(zrows, acc_sh.at[pl.ds(r0, RPW)])
    pltpu.sync_copy(ones_in, rows0)
    plsc.subcore_barrier()

    def start_count(b):
        pltpu.async_copy(rows0, acc_sh.at[dstb[b]], semb[b], add=True)

    def wait_count(b):
        pltpu.make_async_copy(rows0, acc_sh.at[dstb[b]], semb[b]).wait()

    load_idx(0, 0, with_src=False)
    start_count(0)
    load_idx(1, 1, with_src=False)
    start_count(1)

    def batch_b(i, carry):
        g = 2 * i + 2
        wait_count(0)
        load_idx(g, 0, with_src=False)
        start_count(0)
        wait_count(1)
        load_idx(g + 1, 1, with_src=False)
        start_count(1)
        return carry

    lax.fori_loop(0, (NCHUNK - 2) // 2, batch_b, 0)
    wait_count(0)
    wait_count(1)
    plsc.subcore_barrier()
    pltpu.sync_copy(acc_sh.at[pl.ds(r0, RPW)], out_cnts.at[c, pl.ds(r0, RPW)])


_BR = 1000


def _combine_body(lbls_ref, mask_ref, s1_ref, c1_ref, s2_ref, c2_ref, o_ref):
    c1 = jnp.maximum(c1_ref[:, 0:1], 1.0)
    c2 = jnp.maximum(c2_ref[:, 0:1], 1.0)
    mean = (s1_ref[...] / c1 + s2_ref[...] / c2) * 0.5
    o_ref[...] = jnp.where(mask_ref[...] > 0, mean, lbls_ref[...])


def kernel(lbls, no_lbl_idx, knn_sc, knn_fc):
    srcs = jnp.concatenate([knn_sc[0], knn_fc[0]])
    dsts = jnp.concatenate([knn_sc[1], knn_fc[1]])
    zrows = jnp.zeros((RPW, D), jnp.float32)
    ones = jnp.ones((K, D), jnp.float32)
    sums, cnts = _segment_sums(srcs, dsts, lbls, zrows, ones)

    mask2d = no_lbl_idx.astype(jnp.int32).reshape(N, 1)
    return pl.pallas_call(
        _combine_body,
        out_shape=jax.ShapeDtypeStruct((N, D), jnp.float32),
        grid=(N // _BR,),
        in_specs=[
            pl.BlockSpec((_BR, D), lambda i: (i, 0)),
            pl.BlockSpec((_BR, 1), lambda i: (i, 0)),
            pl.BlockSpec((_BR, D), lambda i: (i, 0)),
            pl.BlockSpec((_BR, D), lambda i: (i, 0)),
            pl.BlockSpec((_BR, D), lambda i: (i, 0)),
            pl.BlockSpec((_BR, D), lambda i: (i, 0)),
        ],
        out_specs=pl.BlockSpec((_BR, D), lambda i: (i, 0)),
    )(lbls, mask2d, sums[0], cnts[0], sums[1], cnts[1])


# speedup vs baseline: 2.0570x; 1.6944x over previous
import functools

import jax
import jax.numpy as jnp
from jax import lax
from jax.experimental import pallas as pl
from jax.experimental.pallas import tpu as pltpu
from jax.experimental.pallas import tpu_sc as plsc

N = 10000
E = 320000
D = 128
NC = 2
NS = 16
K = 80
NB = 8
NP = 10240
NCHUNK = 256
EP = NCHUNK * NS * K
NBATCH = NCHUNK // NB
NTAIL = 0
RPW = NP // NS

_mesh = plsc.VectorSubcoreMesh(core_axis_name="c", subcore_axis_name="s")


@functools.partial(
    pl.kernel,
    out_type=(
        jax.ShapeDtypeStruct((NC, NP, D), jnp.float32),
        jax.ShapeDtypeStruct((NC, NP, D), jnp.float32),
    ),
    mesh=_mesh,
    scratch_types=(
        pltpu.VMEM_SHARED((NP, D), jnp.float32),
        pltpu.VMEM((NB, K), jnp.int32),
        pltpu.VMEM((NB, K), jnp.int32),
        pltpu.VMEM((K, D), jnp.float32),
        pltpu.VMEM((K, D), jnp.float32),
        pltpu.SemaphoreType.DMA,
        pltpu.SemaphoreType.DMA,
    ),
)
def _segment_sums(srcs2d, dsts2d, lbls, zrows, ones_in,
                  out_sums, out_cnts,
                  acc_sh, src_b, dst_b, rows0, rows1, sem0, sem1):
    c = lax.axis_index("c")
    s = lax.axis_index("s")
    r0 = pl.multiple_of(s * RPW, 8)
    row_base = c * (NCHUNK * NS) + s * NCHUNK
    rowsb = (rows0, rows1)
    semb = (sem0, sem1)

    def load_batch(i, n, with_src=True):
        roff = pl.multiple_of(row_base + i * NB, 8)
        if with_src:
            pltpu.sync_copy(srcs2d.at[pl.ds(roff, n)], src_b.at[pl.ds(0, n)])
        pltpu.sync_copy(dsts2d.at[pl.ds(roff, n)], dst_b.at[pl.ds(0, n)])

    def start_gather(j, b):
        pltpu.async_copy(lbls.at[src_b.at[j]], rowsb[b], semb[b])

    def finish_scatter(j, b):
        pltpu.make_async_copy(lbls.at[src_b.at[j]], rowsb[b], semb[b]).wait()
        pltpu.sync_copy(rowsb[b], acc_sh.at[dst_b.at[j]], add=True)

    pltpu.sync_copy(zrows, acc_sh.at[pl.ds(r0, RPW)])
    plsc.subcore_barrier()

    def batch_a(i, carry):
        load_batch(i, NB)
        start_gather(0, 0)
        for j in range(1, NB):
            start_gather(j, j % 2)
            finish_scatter(j - 1, (j - 1) % 2)
        finish_scatter(NB - 1, (NB - 1) % 2)
        return carry

    lax.fori_loop(0, NBATCH, batch_a, 0)
    if NTAIL:
        load_batch(NBATCH, NTAIL)
        start_gather(0, 0)
        for j in range(1, NTAIL):
            start_gather(j, j % 2)
            finish_scatter(j - 1, (j - 1) % 2)
        finish_scatter(NTAIL - 1, (NTAIL - 1) % 2)
    plsc.subcore_barrier()
    pltpu.sync_copy(acc_sh.at[pl.ds(r0, RPW)], out_sums.at[c, pl.ds(r0, RPW)])
    plsc.subcore_barrier()

    pltpu.sync_copy(zrows, acc_sh.at[pl.ds(r0, RPW)])
    pltpu.sync_copy(ones_in, rows0)
    plsc.subcore_barrier()

    def batch_b(i, carry):
        load_batch(i, NB, with_src=False)
        for j in range(NB):
            pltpu.sync_copy(rows0, acc_sh.at[dst_b.at[j]], add=True)
        return carry

    lax.fori_loop(0, NBATCH, batch_b, 0)
    if NTAIL:
        load_batch(NBATCH, NTAIL, with_src=False)
        for j in range(NTAIL):
            pltpu.sync_copy(rows0, acc_sh.at[dst_b.at[j]], add=True)
    plsc.subcore_barrier()
    pltpu.sync_copy(acc_sh.at[pl.ds(r0, RPW)], out_cnts.at[c, pl.ds(r0, RPW)])


_BR = 1000


def _combine_body(lbls_ref, mask_ref, s1_ref, c1_ref, s2_ref, c2_ref, o_ref):
    c1 = jnp.maximum(c1_ref[:, 0:1], 1.0)
    c2 = jnp.maximum(c2_ref[:, 0:1], 1.0)
    mean = (s1_ref[...] / c1 + s2_ref[...] / c2) * 0.5
    o_ref[...] = jnp.where(mask_ref[...] > 0, mean, lbls_ref[...])


def _pad_set(knn):
    npad = EP - E
    ar = jnp.arange(npad, dtype=jnp.int32)
    src = jnp.concatenate([knn[0], ar % N])
    dst = jnp.concatenate([knn[1], N + ar % (NP - N)])
    return src, dst


def kernel(lbls, no_lbl_idx, knn_sc, knn_fc):
    s1, d1 = _pad_set(knn_sc)
    s2, d2 = _pad_set(knn_fc)
    srcs2d = jnp.concatenate([s1, s2]).reshape(NC * NS * NCHUNK, K)
    dsts2d = jnp.concatenate([d1, d2]).reshape(NC * NS * NCHUNK, K)
    zrows = jnp.zeros((RPW, D), jnp.float32)
    ones = jnp.ones((K, D), jnp.float32)
    sums, cnts = _segment_sums(srcs2d, dsts2d, lbls, zrows, ones)

    mask2d = no_lbl_idx.astype(jnp.int32).reshape(N, 1)
    return pl.pallas_call(
        _combine_body,
        out_shape=jax.ShapeDtypeStruct((N, D), jnp.float32),
        grid=(N // _BR,),
        in_specs=[
            pl.BlockSpec((_BR, D), lambda i: (i, 0)),
            pl.BlockSpec((_BR, 1), lambda i: (i, 0)),
            pl.BlockSpec((_BR, D), lambda i: (i, 0)),
            pl.BlockSpec((_BR, D), lambda i: (i, 0)),
            pl.BlockSpec((_BR, D), lambda i: (i, 0)),
            pl.BlockSpec((_BR, D), lambda i: (i, 0)),
        ],
        out_specs=pl.BlockSpec((_BR, D), lambda i: (i, 0)),
    )(lbls, mask2d, sums[0], cnts[0], sums[1], cnts[1])


# speedup vs baseline: 2.2320x; 1.0851x over previous
import functools

import jax
import jax.numpy as jnp
from jax import lax
from jax.experimental import pallas as pl
from jax.experimental.pallas import tpu as pltpu
from jax.experimental.pallas import tpu_sc as plsc

N = 10000
E = 320000
D = 128
NC = 2
NS = 16
K = 80
NB = 16
NP = 10240
NCHUNK = 256
EP = NCHUNK * NS * K
NBATCH = NCHUNK // NB
NTAIL = 0
RPW = NP // NS

_mesh = plsc.VectorSubcoreMesh(core_axis_name="c", subcore_axis_name="s")


@functools.partial(
    pl.kernel,
    out_type=(
        jax.ShapeDtypeStruct((NC, NP, D), jnp.float32),
        jax.ShapeDtypeStruct((NC, NP, D), jnp.float32),
    ),
    mesh=_mesh,
    scratch_types=(
        pltpu.VMEM_SHARED((NP, D), jnp.float32),
        pltpu.VMEM((NB, K), jnp.int32),
        pltpu.VMEM((NB, K), jnp.int32),
        pltpu.VMEM((K, D), jnp.float32),
        pltpu.VMEM((K, D), jnp.float32),
        pltpu.SemaphoreType.DMA,
        pltpu.SemaphoreType.DMA,
    ),
)
def _segment_sums(srcs2d, dsts2d, lbls, zrows, ones_in,
                  out_sums, out_cnts,
                  acc_sh, src_b, dst_b, rows0, rows1, sem0, sem1):
    c = lax.axis_index("c")
    s = lax.axis_index("s")
    r0 = pl.multiple_of(s * RPW, 8)
    row_base = c * (NCHUNK * NS) + s * NCHUNK
    rowsb = (rows0, rows1)
    semb = (sem0, sem1)

    def load_batch(i, n, with_src=True):
        roff = pl.multiple_of(row_base + i * NB, 8)
        if with_src:
            pltpu.sync_copy(srcs2d.at[pl.ds(roff, n)], src_b.at[pl.ds(0, n)])
        pltpu.sync_copy(dsts2d.at[pl.ds(roff, n)], dst_b.at[pl.ds(0, n)])

    def start_gather(j, b):
        pltpu.async_copy(lbls.at[src_b.at[j]], rowsb[b], semb[b])

    def finish_scatter(j, b):
        pltpu.make_async_copy(lbls.at[src_b.at[j]], rowsb[b], semb[b]).wait()
        pltpu.sync_copy(rowsb[b], acc_sh.at[dst_b.at[j]], add=True)

    pltpu.sync_copy(zrows, acc_sh.at[pl.ds(r0, RPW)])
    plsc.subcore_barrier()

    def batch_a(i, carry):
        load_batch(i, NB)
        start_gather(0, 0)
        for j in range(1, NB):
            start_gather(j, j % 2)
            finish_scatter(j - 1, (j - 1) % 2)
        finish_scatter(NB - 1, (NB - 1) % 2)
        return carry

    lax.fori_loop(0, NBATCH, batch_a, 0)
    if NTAIL:
        load_batch(NBATCH, NTAIL)
        start_gather(0, 0)
        for j in range(1, NTAIL):
            start_gather(j, j % 2)
            finish_scatter(j - 1, (j - 1) % 2)
        finish_scatter(NTAIL - 1, (NTAIL - 1) % 2)
    plsc.subcore_barrier()
    pltpu.sync_copy(acc_sh.at[pl.ds(r0, RPW)], out_sums.at[c, pl.ds(r0, RPW)])
    plsc.subcore_barrier()

    pltpu.sync_copy(zrows, acc_sh.at[pl.ds(r0, RPW)])
    pltpu.sync_copy(ones_in, rows0)
    plsc.subcore_barrier()

    DEPTH = 4

    def start_count(j):
        pltpu.async_copy(rows0, acc_sh.at[dst_b.at[j]], sem0, add=True)

    def wait_count(j):
        pltpu.make_async_copy(rows0, acc_sh.at[dst_b.at[j]], sem0).wait()

    def batch_b(i, carry):
        load_batch(i, NB, with_src=False)
        for j in range(NB):
            if j >= DEPTH:
                wait_count(j - DEPTH)
            start_count(j)
        for j in range(NB - DEPTH, NB):
            wait_count(j)
        return carry

    lax.fori_loop(0, NBATCH, batch_b, 0)
    plsc.subcore_barrier()
    pltpu.sync_copy(acc_sh.at[pl.ds(r0, RPW)], out_cnts.at[c, pl.ds(r0, RPW)])


_BR = 1000


def _combine_body(lbls_ref, mask_ref, s1_ref, c1_ref, s2_ref, c2_ref, o_ref):
    c1 = jnp.maximum(c1_ref[:, 0:1], 1.0)
    c2 = jnp.maximum(c2_ref[:, 0:1], 1.0)
    mean = (s1_ref[...] / c1 + s2_ref[...] / c2) * 0.5
    o_ref[...] = jnp.where(mask_ref[...] > 0, mean, lbls_ref[...])


def _pad_set(knn):
    npad = EP - E
    ar = jnp.arange(npad, dtype=jnp.int32)
    src = jnp.concatenate([knn[0], ar % N])
    dst = jnp.concatenate([knn[1], N + ar % (NP - N)])
    return src, dst


def kernel(lbls, no_lbl_idx, knn_sc, knn_fc):
    s1, d1 = _pad_set(knn_sc)
    s2, d2 = _pad_set(knn_fc)
    srcs2d = jnp.concatenate([s1, s2]).reshape(NC * NS * NCHUNK, K)
    dsts2d = jnp.concatenate([d1, d2]).reshape(NC * NS * NCHUNK, K)
    zrows = jnp.zeros((RPW, D), jnp.float32)
    ones = jnp.ones((K, D), jnp.float32)
    sums, cnts = _segment_sums(srcs2d, dsts2d, lbls, zrows, ones)

    mask2d = no_lbl_idx.astype(jnp.int32).reshape(N, 1)
    return pl.pallas_call(
        _combine_body,
        out_shape=jax.ShapeDtypeStruct((N, D), jnp.float32),
        grid=(N // _BR,),
        in_specs=[
            pl.BlockSpec((_BR, D), lambda i: (i, 0)),
            pl.BlockSpec((_BR, 1), lambda i: (i, 0)),
            pl.BlockSpec((_BR, D), lambda i: (i, 0)),
            pl.BlockSpec((_BR, D), lambda i: (i, 0)),
            pl.BlockSpec((_BR, D), lambda i: (i, 0)),
            pl.BlockSpec((_BR, D), lambda i: (i, 0)),
        ],
        out_specs=pl.BlockSpec((_BR, D), lambda i: (i, 0)),
    )(lbls, mask2d, sums[0], cnts[0], sums[1], cnts[1])


# speedup vs baseline: 2.2333x; 1.0006x over previous
import functools

import jax
import jax.numpy as jnp
from jax import lax
from jax.experimental import pallas as pl
from jax.experimental.pallas import tpu as pltpu
from jax.experimental.pallas import tpu_sc as plsc

N = 10000
E = 320000
D = 128
NC = 2
NS = 16
K = 80
NB = 16
NP = 10240
NCHUNK = 256
EP = NCHUNK * NS * K
NBATCH = NCHUNK // NB
NTAIL = 0
RPW = NP // NS

_mesh = plsc.VectorSubcoreMesh(core_axis_name="c", subcore_axis_name="s")


@functools.partial(
    pl.kernel,
    out_type=(
        jax.ShapeDtypeStruct((NC, NP, D), jnp.float32),
        jax.ShapeDtypeStruct((NC, NP, D), jnp.float32),
    ),
    mesh=_mesh,
    scratch_types=(
        pltpu.VMEM_SHARED((NP, D), jnp.float32),
        pltpu.VMEM((NB, K), jnp.int32),
        pltpu.VMEM((NB, K), jnp.int32),
        pltpu.VMEM((K, D), jnp.float32),
        pltpu.VMEM((K, D), jnp.float32),
        pltpu.SemaphoreType.DMA,
        pltpu.SemaphoreType.DMA,
        pltpu.SemaphoreType.DMA,
    ),
)
def _segment_sums(srcs2d, dsts2d, lbls, zrows, ones_in,
                  out_sums, out_cnts,
                  acc_sh, src_b, dst_b, rows0, rows1, sem0, sem1, sem2):
    c = lax.axis_index("c")
    s = lax.axis_index("s")
    r0 = pl.multiple_of(s * RPW, 8)
    row_base = c * (NCHUNK * NS) + s * NCHUNK
    rowsb = (rows0, rows1)
    semb = (sem0, sem1)

    def load_batch(i, n, with_src=True):
        roff = pl.multiple_of(row_base + i * NB, 8)
        if with_src:
            pltpu.sync_copy(srcs2d.at[pl.ds(roff, n)], src_b.at[pl.ds(0, n)])
        pltpu.sync_copy(dsts2d.at[pl.ds(roff, n)], dst_b.at[pl.ds(0, n)])

    def start_gather(j, b):
        pltpu.async_copy(lbls.at[src_b.at[j]], rowsb[b], semb[b])

    def wait_gather(j, b):
        pltpu.make_async_copy(lbls.at[src_b.at[j]], rowsb[b], semb[b]).wait()

    def start_scatter(j, b):
        pltpu.async_copy(rowsb[b], acc_sh.at[dst_b.at[j]], sem2, add=True)

    def wait_scatter(j, b):
        pltpu.make_async_copy(rowsb[b], acc_sh.at[dst_b.at[j]], sem2).wait()

    pltpu.sync_copy(zrows, acc_sh.at[pl.ds(r0, RPW)])
    plsc.subcore_barrier()

    def batch_a(i, carry):
        load_batch(i, NB)
        for j in range(NB):
            b = j % 2
            if j >= 2:
                wait_scatter(j - 2, b)
            start_gather(j, b)
            if j >= 1:
                wait_gather(j - 1, 1 - b)
                start_scatter(j - 1, 1 - b)
        wait_gather(NB - 1, (NB - 1) % 2)
        start_scatter(NB - 1, (NB - 1) % 2)
        wait_scatter(NB - 2, (NB - 2) % 2)
        wait_scatter(NB - 1, (NB - 1) % 2)
        return carry

    lax.fori_loop(0, NBATCH, batch_a, 0)
    if NTAIL:
        load_batch(NBATCH, NTAIL)
        start_gather(0, 0)
        for j in range(1, NTAIL):
            start_gather(j, j % 2)
            finish_scatter(j - 1, (j - 1) % 2)
        finish_scatter(NTAIL - 1, (NTAIL - 1) % 2)
    plsc.subcore_barrier()
    pltpu.sync_copy(acc_sh.at[pl.ds(r0, RPW)], out_sums.at[c, pl.ds(r0, RPW)])
    plsc.subcore_barrier()

    pltpu.sync_copy(zrows, acc_sh.at[pl.ds(r0, RPW)])
    pltpu.sync_copy(ones_in, rows0)
    plsc.subcore_barrier()

    DEPTH = 4

    def start_count(j):
        pltpu.async_copy(rows0, acc_sh.at[dst_b.at[j]], sem0, add=True)

    def wait_count(j):
        pltpu.make_async_copy(rows0, acc_sh.at[dst_b.at[j]], sem0).wait()

    def batch_b(i, carry):
        load_batch(i, NB, with_src=False)
        for j in range(NB):
            if j >= DEPTH:
                wait_count(j - DEPTH)
            start_count(j)
        for j in range(NB - DEPTH, NB):
            wait_count(j)
        return carry

    lax.fori_loop(0, NBATCH, batch_b, 0)
    plsc.subcore_barrier()
    pltpu.sync_copy(acc_sh.at[pl.ds(r0, RPW)], out_cnts.at[c, pl.ds(r0, RPW)])


_BR = 1000


def _combine_body(lbls_ref, mask_ref, s1_ref, c1_ref, s2_ref, c2_ref, o_ref):
    c1 = jnp.maximum(c1_ref[:, 0:1], 1.0)
    c2 = jnp.maximum(c2_ref[:, 0:1], 1.0)
    mean = (s1_ref[...] / c1 + s2_ref[...] / c2) * 0.5
    o_ref[...] = jnp.where(mask_ref[...] > 0, mean, lbls_ref[...])


def _pad_set(knn):
    npad = EP - E
    ar = jnp.arange(npad, dtype=jnp.int32)
    src = jnp.concatenate([knn[0], ar % N])
    dst = jnp.concatenate([knn[1], N + ar % (NP - N)])
    return src, dst


def kernel(lbls, no_lbl_idx, knn_sc, knn_fc):
    s1, d1 = _pad_set(knn_sc)
    s2, d2 = _pad_set(knn_fc)
    srcs2d = jnp.concatenate([s1, s2]).reshape(NC * NS * NCHUNK, K)
    dsts2d = jnp.concatenate([d1, d2]).reshape(NC * NS * NCHUNK, K)
    zrows = jnp.zeros((RPW, D), jnp.float32)
    ones = jnp.ones((K, D), jnp.float32)
    sums, cnts = _segment_sums(srcs2d, dsts2d, lbls, zrows, ones)

    mask2d = no_lbl_idx.astype(jnp.int32).reshape(N, 1)
    return pl.pallas_call(
        _combine_body,
        out_shape=jax.ShapeDtypeStruct((N, D), jnp.float32),
        grid=(N // _BR,),
        in_specs=[
            pl.BlockSpec((_BR, D), lambda i: (i, 0)),
            pl.BlockSpec((_BR, 1), lambda i: (i, 0)),
            pl.BlockSpec((_BR, D), lambda i: (i, 0)),
            pl.BlockSpec((_BR, D), lambda i: (i, 0)),
            pl.BlockSpec((_BR, D), lambda i: (i, 0)),
            pl.BlockSpec((_BR, D), lambda i: (i, 0)),
        ],
        out_specs=pl.BlockSpec((_BR, D), lambda i: (i, 0)),
    )(lbls, mask2d, sums[0], cnts[0], sums[1], cnts[1])
